# baseline (speedup 1.0000x reference)
import dataclasses
import functools

import jax
import jax.numpy as jnp
from jax import lax
from jax.experimental import pallas as pl
from jax.experimental.pallas import tpu as pltpu
from jax.experimental.pallas import tpu_sc as plsc

B, S, D = 32, 128, 1024
K_KEYS = 100000
TOPK = 32


_KC = 2048
_NKC = (K_KEYS + _KC - 1) // _KC


def _sims_body(q_ref, k_ref, o_ref):
    o_ref[...] = lax.dot_general(
        q_ref[...], k_ref[...], (((1,), (1,)), ((), ())),
        preferred_element_type=jnp.float32,
        precision=lax.Precision.HIGHEST,
    )


def _sims(q, keys):
    return pl.pallas_call(
        _sims_body,
        grid=(_NKC,),
        in_specs=[
            pl.BlockSpec((B, D), lambda i: (0, 0)),
            pl.BlockSpec((_KC, D), lambda i: (i, 0)),
        ],
        out_specs=pl.BlockSpec((B, _KC), lambda i: (0, i)),
        out_shape=jax.ShapeDtypeStruct((B, K_KEYS), jnp.float32),
    )(q, keys)


_RB = 512


def _y_body(x_ref, w_ref, b_ref, o_ref):
    o_ref[...] = lax.dot_general(
        x_ref[...], w_ref[...], (((1,), (1,)), ((), ())),
        preferred_element_type=jnp.float32,
        precision=lax.Precision.HIGHEST,
    ) + b_ref[...]


def _linear(x2, W, b2):
    return pl.pallas_call(
        _y_body,
        grid=(B * S // _RB,),
        in_specs=[
            pl.BlockSpec((_RB, D), lambda i: (i, 0)),
            pl.BlockSpec((D, D), lambda i: (0, 0)),
            pl.BlockSpec((1, D), lambda i: (0, 0)),
        ],
        out_specs=pl.BlockSpec((_RB, D), lambda i: (i, 0)),
        out_shape=jax.ShapeDtypeStruct((B * S, D), jnp.float32),
    )(x2, W, b2)


_CH = 10000
_NCHUNK = K_KEYS // _CH
_L = 16

_NEG = -3.0e38


def _merge16(xv, xi, yv, yi):
    rv = lax.rev(yv, (0,))
    ri = lax.rev(yi, (0,))
    m = xv >= rv
    hi = jnp.maximum(xv, rv)
    lo = jnp.minimum(xv, rv)
    hii = jnp.where(m, xi, ri)
    loi = jnp.where(m, ri, xi)
    hi_s, hii_s = plsc.sort_key_val(hi, hii, descending=True)
    lo_s, loi_s = plsc.sort_key_val(lo, loi, descending=True)
    return hi_s, hii_s, lo_s, loi_s


def _sc_body(sims_hbm, keys_hbm, nbr_hbm,
             buf0, buf1, s0v, s0i, s1v, s1i, tref, idxr, nbr, sem0, sem1,
             gsem, osem):
    wid = lax.axis_index("s") * 2 + lax.axis_index("c")
    row0 = wid * K_KEYS

    s0v[...] = jnp.full((_L,), _NEG, jnp.float32)
    s1v[...] = jnp.full((_L,), _NEG, jnp.float32)
    s0i[...] = jnp.zeros((_L,), jnp.int32)
    s1i[...] = jnp.zeros((_L,), jnp.int32)
    tref[...] = jnp.full((_L,), _NEG, jnp.float32)

    lane = lax.iota(jnp.int32, _L)

    def insert(v, base):
        giv = lane + base
        sv, si = plsc.sort_key_val(v, giv, descending=True)
        a_v, a_i, lo_v, lo_i = _merge16(s0v[...], s0i[...], sv, si)
        s0v[...] = a_v
        s0i[...] = a_i
        b_v, b_i, _, _ = _merge16(s1v[...], s1i[...], lo_v, lo_i)
        s1v[...] = b_v
        s1i[...] = b_i
        tref[...] = jnp.zeros((_L,), jnp.float32) + jnp.min(b_v)

    def scan_chunk(buf, base):
        @pl.loop(0, _CH, step=_L)
        def _(i):
            v = buf[pl.ds(i, _L)]
            hit = jnp.any(v > tref[...])

            @pl.when(hit)
            def _():
                insert(v, base + i)

    cp0 = pltpu.async_copy(sims_hbm.at[pl.ds(row0, _CH)], buf0, sem0)
    for c in range(0, _NCHUNK, 2):
        if c + 1 < _NCHUNK:
            cp1 = pltpu.async_copy(
                sims_hbm.at[pl.ds(row0 + (c + 1) * _CH, _CH)], buf1, sem1)
        cp0.wait()
        scan_chunk(buf0, c * _CH)
        if c + 2 < _NCHUNK:
            cp0 = pltpu.async_copy(
                sims_hbm.at[pl.ds(row0 + (c + 2) * _CH, _CH)], buf0, sem0)
        if c + 1 < _NCHUNK:
            cp1.wait()
            scan_chunk(buf1, (c + 1) * _CH)

    idxr[pl.ds(0, _L)] = s0i[...]
    idxr[pl.ds(_L, _L)] = s1i[...]
    pltpu.async_copy(keys_hbm.at[idxr], nbr, gsem).wait()
    pltpu.async_copy(nbr, nbr_hbm.at[pl.ds(wid * TOPK, TOPK)], osem).wait()


def _sc_compiler_params():
    cp = pltpu.CompilerParams()
    if "needs_layout_passes" in pltpu.CompilerParams.__dataclass_fields__:
        cp = dataclasses.replace(cp, needs_layout_passes=False)
    return cp


def _sc_topk_gather(sims_flat, keys):
    mesh = plsc.VectorSubcoreMesh(core_axis_name="c", subcore_axis_name="s")
    kern = functools.partial(
        pl.kernel,
        compiler_params=_sc_compiler_params(),
        out_type=jax.ShapeDtypeStruct((B * TOPK, D), jnp.float32),
        mesh=mesh,
        scratch_types=[
            pltpu.VMEM((_CH,), jnp.float32),
            pltpu.VMEM((_CH,), jnp.float32),
            pltpu.VMEM((_L,), jnp.float32),
            pltpu.VMEM((_L,), jnp.int32),
            pltpu.VMEM((_L,), jnp.float32),
            pltpu.VMEM((_L,), jnp.int32),
            pltpu.VMEM((_L,), jnp.float32),
            pltpu.VMEM((TOPK,), jnp.int32),
            pltpu.VMEM((TOPK, D), jnp.float32),
            pltpu.SemaphoreType.DMA,
            pltpu.SemaphoreType.DMA,
            pltpu.SemaphoreType.DMA,
            pltpu.SemaphoreType.DMA,
        ],
    )(_sc_body)
    return kern(sims_flat, keys)


def _fin_body(y0_ref, nbr_ref, m_ref, o_ref):
    w = jax.nn.softmax(m_ref[0, :]) * jnp.float32(1.0 / TOPK)
    nmean = jnp.sum(nbr_ref[...] * w[None, :, None], axis=1)
    o_ref[...] = y0_ref[...] + nmean


def _finalize(y0, nbr3, maskr):
    return pl.pallas_call(
        _fin_body,
        in_specs=[
            pl.BlockSpec((B, D), lambda: (0, 0)),
            pl.BlockSpec((B, TOPK, D), lambda: (0, 0, 0)),
            pl.BlockSpec((1, TOPK), lambda: (0, 0)),
        ],
        out_specs=pl.BlockSpec((B, D), lambda: (0, 0)),
        out_shape=jax.ShapeDtypeStruct((B, D), jnp.float32),
    )(y0, nbr3, maskr)


def kernel(x, keys, W, b, mask):
    x2 = x.reshape(B * S, D)
    q = x[:, 0, :]
    sims = _sims(q, keys)
    y2 = _linear(x2, W, b.reshape(1, D))
    nbr = _sc_topk_gather(sims.reshape(B * K_KEYS), keys)
    y = y2.reshape(B, S, D)
    y0 = _finalize(y[:, 0, :], nbr.reshape(B, TOPK, D), mask.reshape(1, TOPK))
    return y.at[:, 0, :].set(y0)


# speedup vs baseline: 2.9775x; 2.9775x over previous
import dataclasses
import functools

import jax
import jax.numpy as jnp
from jax import lax
from jax.experimental import pallas as pl
from jax.experimental.pallas import tpu as pltpu
from jax.experimental.pallas import tpu_sc as plsc

B, S, D = 32, 128, 1024
K_KEYS = 100000
TOPK = 32


_KC = 2048
_NKC = (K_KEYS + _KC - 1) // _KC
_KPAD = _NKC * _KC


def _sims_body(q_ref, k_ref, o_ref):
    s = lax.dot_general(
        q_ref[...], k_ref[...], (((1,), (1,)), ((), ())),
        preferred_element_type=jnp.float32,
    )
    i = pl.program_id(0)

    @pl.when(i == _NKC - 1)
    def _():
        col = i * _KC + lax.broadcasted_iota(jnp.int32, (B, _KC), 1)
        o_ref[...] = jnp.where(col >= K_KEYS, jnp.float32(_NEG), s)

    @pl.when(i != _NKC - 1)
    def _():
        o_ref[...] = s


def _sims(q, keys):
    return pl.pallas_call(
        _sims_body,
        grid=(_NKC,),
        in_specs=[
            pl.BlockSpec((B, D), lambda i: (0, 0)),
            pl.BlockSpec((_KC, D), lambda i: (i, 0)),
        ],
        out_specs=pl.BlockSpec((B, _KC), lambda i: (0, i)),
        out_shape=jax.ShapeDtypeStruct((B, _KPAD), jnp.float32),
    )(q, keys)


_RB = 512


def _y_body(x_ref, w_ref, b_ref, o_ref):
    o_ref[...] = lax.dot_general(
        x_ref[...], w_ref[...], (((1,), (1,)), ((), ())),
        preferred_element_type=jnp.float32,
    ) + b_ref[...]


def _linear(x2, W, b2):
    return pl.pallas_call(
        _y_body,
        grid=(B * S // _RB,),
        in_specs=[
            pl.BlockSpec((_RB, D), lambda i: (i, 0)),
            pl.BlockSpec((D, D), lambda i: (0, 0)),
            pl.BlockSpec((1, D), lambda i: (0, 0)),
        ],
        out_specs=pl.BlockSpec((_RB, D), lambda i: (i, 0)),
        out_shape=jax.ShapeDtypeStruct((B * S, D), jnp.float32),
    )(x2, W, b2)


_CH = 12544
_NCHUNK = _KPAD // _CH
_L = 16
_G = 8

_NEG = -3.0e38


def _merge16(xv, xi, yv, yi):
    rv = lax.rev(yv, (0,))
    ri = lax.rev(yi, (0,))
    m = xv >= rv
    hi = jnp.maximum(xv, rv)
    lo = jnp.minimum(xv, rv)
    hii = jnp.where(m, xi, ri)
    loi = jnp.where(m, ri, xi)
    hi_s, hii_s = plsc.sort_key_val(hi, hii, descending=True)
    lo_s, loi_s = plsc.sort_key_val(lo, loi, descending=True)
    return hi_s, hii_s, lo_s, loi_s


def _sc_body(sims_hbm, keys_hbm, nbr_hbm,
             buf0, buf1, s0v, s0i, s1v, s1i, tref, idxr, nbr, sem0, sem1,
             gsem, osem):
    wid = lax.axis_index("s") * 2 + lax.axis_index("c")
    row0 = wid * _KPAD

    s0v[...] = jnp.full((_L,), _NEG, jnp.float32)
    s1v[...] = jnp.full((_L,), _NEG, jnp.float32)
    s0i[...] = jnp.zeros((_L,), jnp.int32)
    s1i[...] = jnp.zeros((_L,), jnp.int32)
    tref[...] = jnp.full((_L,), _NEG, jnp.float32)

    lane = lax.iota(jnp.int32, _L)

    def insert(v, base):
        giv = lane + base
        sv, si = plsc.sort_key_val(v, giv, descending=True)
        a_v, a_i, lo_v, lo_i = _merge16(s0v[...], s0i[...], sv, si)
        s0v[...] = a_v
        s0i[...] = a_i
        b_v, b_i, _, _ = _merge16(s1v[...], s1i[...], lo_v, lo_i)
        s1v[...] = b_v
        s1i[...] = b_i
        tref[...] = jnp.zeros((_L,), jnp.float32) + jnp.min(b_v)

    def scan_chunk(buf, base):
        @pl.loop(0, _CH, step=_L * _G)
        def _(i):
            t = tref[...]
            m = buf[pl.ds(i, _L)]
            for u in range(1, _G):
                m = jnp.maximum(m, buf[pl.ds(i + u * _L, _L)])
            hit = jnp.any(m > t)

            @pl.when(hit)
            def _():
                @pl.loop(0, _G * _L, step=_L)
                def _(j):
                    v = buf[pl.ds(i + j, _L)]
                    hit2 = jnp.any(v > tref[...])

                    @pl.when(hit2)
                    def _():
                        insert(v, base + i + j)

    cp0 = pltpu.async_copy(sims_hbm.at[pl.ds(row0, _CH)], buf0, sem0)
    for c in range(0, _NCHUNK, 2):
        if c + 1 < _NCHUNK:
            cp1 = pltpu.async_copy(
                sims_hbm.at[pl.ds(row0 + (c + 1) * _CH, _CH)], buf1, sem1)
        cp0.wait()
        scan_chunk(buf0, c * _CH)
        if c + 2 < _NCHUNK:
            cp0 = pltpu.async_copy(
                sims_hbm.at[pl.ds(row0 + (c + 2) * _CH, _CH)], buf0, sem0)
        if c + 1 < _NCHUNK:
            cp1.wait()
            scan_chunk(buf1, (c + 1) * _CH)

    idxr[pl.ds(0, _L)] = s0i[...]
    idxr[pl.ds(_L, _L)] = s1i[...]
    pltpu.async_copy(keys_hbm.at[idxr], nbr, gsem).wait()
    pltpu.async_copy(nbr, nbr_hbm.at[pl.ds(wid * TOPK, TOPK)], osem).wait()


def _sc_compiler_params():
    cp = pltpu.CompilerParams()
    if "needs_layout_passes" in pltpu.CompilerParams.__dataclass_fields__:
        cp = dataclasses.replace(cp, needs_layout_passes=False)
    return cp


def _sc_topk_gather(sims_flat, keys):
    mesh = plsc.VectorSubcoreMesh(core_axis_name="c", subcore_axis_name="s")
    kern = functools.partial(
        pl.kernel,
        compiler_params=_sc_compiler_params(),
        out_type=jax.ShapeDtypeStruct((B * TOPK, D), jnp.float32),
        mesh=mesh,
        scratch_types=[
            pltpu.VMEM((_CH,), jnp.float32),
            pltpu.VMEM((_CH,), jnp.float32),
            pltpu.VMEM((_L,), jnp.float32),
            pltpu.VMEM((_L,), jnp.int32),
            pltpu.VMEM((_L,), jnp.float32),
            pltpu.VMEM((_L,), jnp.int32),
            pltpu.VMEM((_L,), jnp.float32),
            pltpu.VMEM((TOPK,), jnp.int32),
            pltpu.VMEM((TOPK, D), jnp.float32),
            pltpu.SemaphoreType.DMA,
            pltpu.SemaphoreType.DMA,
            pltpu.SemaphoreType.DMA,
            pltpu.SemaphoreType.DMA,
        ],
    )(_sc_body)
    return kern(sims_flat, keys)


def _fin_body(y0_ref, nbr_ref, m_ref, o_ref):
    w = jax.nn.softmax(m_ref[0, :]) * jnp.float32(1.0 / TOPK)
    nmean = jnp.sum(nbr_ref[...] * w[None, :, None], axis=1)
    o_ref[...] = y0_ref[...] + nmean


def _finalize(y0, nbr3, maskr):
    return pl.pallas_call(
        _fin_body,
        in_specs=[
            pl.BlockSpec((B, D), lambda: (0, 0)),
            pl.BlockSpec((B, TOPK, D), lambda: (0, 0, 0)),
            pl.BlockSpec((1, TOPK), lambda: (0, 0)),
        ],
        out_specs=pl.BlockSpec((B, D), lambda: (0, 0)),
        out_shape=jax.ShapeDtypeStruct((B, D), jnp.float32),
    )(y0, nbr3, maskr)


def kernel(x, keys, W, b, mask):
    x2 = x.reshape(B * S, D)
    q = x[:, 0, :]
    sims = _sims(q, keys)
    y2 = _linear(x2, W, b.reshape(1, D))
    nbr = _sc_topk_gather(sims.reshape(B * _KPAD), keys)
    y = y2.reshape(B, S, D)
    y0 = _finalize(y[:, 0, :], nbr.reshape(B, TOPK, D), mask.reshape(1, TOPK))
    return y.at[:, 0, :].set(y0)


# speedup vs baseline: 3.5186x; 1.1817x over previous
import dataclasses
import functools

import jax
import jax.numpy as jnp
from jax import lax
from jax.experimental import pallas as pl
from jax.experimental.pallas import tpu as pltpu
from jax.experimental.pallas import tpu_sc as plsc

B, S, D = 32, 128, 1024
K_KEYS = 100000
TOPK = 32


_KC = 2048
_NKC = (K_KEYS + _KC - 1) // _KC
_KPAD = _NKC * _KC


def _sims_body(q_ref, k_ref, o_ref):
    s = lax.dot_general(
        q_ref[...], k_ref[...], (((1,), (1,)), ((), ())),
        preferred_element_type=jnp.float32,
    )
    i = pl.program_id(0)

    @pl.when(i == _NKC - 1)
    def _():
        col = i * _KC + lax.broadcasted_iota(jnp.int32, (B, _KC), 1)
        o_ref[...] = jnp.where(col >= K_KEYS, jnp.float32(_NEG), s)

    @pl.when(i != _NKC - 1)
    def _():
        o_ref[...] = s


def _sims(q, keys):
    return pl.pallas_call(
        _sims_body,
        grid=(_NKC,),
        in_specs=[
            pl.BlockSpec((B, D), lambda i: (0, 0)),
            pl.BlockSpec((_KC, D), lambda i: (i, 0)),
        ],
        out_specs=pl.BlockSpec((B, _KC), lambda i: (0, i)),
        out_shape=jax.ShapeDtypeStruct((B, _KPAD), jnp.float32),
    )(q, keys)


_RB = 512


def _y_body(x_ref, w_ref, b_ref, o_ref):
    o_ref[...] = lax.dot_general(
        x_ref[...], w_ref[...], (((1,), (1,)), ((), ())),
        preferred_element_type=jnp.float32,
    ) + b_ref[...]


def _linear(x2, W, b2):
    return pl.pallas_call(
        _y_body,
        grid=(B * S // _RB,),
        in_specs=[
            pl.BlockSpec((_RB, D), lambda i: (i, 0)),
            pl.BlockSpec((D, D), lambda i: (0, 0)),
            pl.BlockSpec((1, D), lambda i: (0, 0)),
        ],
        out_specs=pl.BlockSpec((_RB, D), lambda i: (i, 0)),
        out_shape=jax.ShapeDtypeStruct((B * S, D), jnp.float32),
    )(x2, W, b2)


_L = 16
_SB = 128
_SG = 2048
_NSG = _KPAD // _SG
_CH = 7 * _SG
_NCHUNK = _KPAD // _CH

_NEG = -3.0e38


def _bcast(s):
    return jnp.zeros((_L,), jnp.float32) + s


def _merge16(xv, xi, yv, yi):
    rv = lax.rev(yv, (0,))
    ri = lax.rev(yi, (0,))
    m = xv >= rv
    hi = jnp.maximum(xv, rv)
    lo = jnp.minimum(xv, rv)
    hii = jnp.where(m, xi, ri)
    loi = jnp.where(m, ri, xi)
    hi_s, hii_s = plsc.sort_key_val(hi, hii, descending=True)
    lo_s, loi_s = plsc.sort_key_val(lo, loi, descending=True)
    return hi_s, hii_s, lo_s, loi_s


def _sc_body(sims_hbm, keys_hbm, nbr_hbm,
             rowbuf, sum1, sum2, gbuf, idxr,
             s0v, s0i, s1v, s1i, tref, t0ref, w0, w1,
             csem0, csem1, csem2, csem3, csem4, csem5, csem6, gsem, osem):
    wid = lax.axis_index("s") * 2 + lax.axis_index("c")
    row0 = wid * _KPAD
    csems = [csem0, csem1, csem2, csem3, csem4, csem5, csem6]

    cps = [
        pltpu.async_copy(
            sims_hbm.at[pl.ds(row0 + c * _CH, _CH)],
            rowbuf.at[pl.ds(c * _CH, _CH)],
            csems[c],
        )
        for c in range(_NCHUNK)
    ]

    for c in range(_NCHUNK):
        cps[c].wait()

        @pl.loop(0, _CH // _SG)
        def _(s):
            sgi = c * (_CH // _SG) + s
            sg0 = sgi * _SG
            macc = None
            for j in range(_SG // _SB):
                m = rowbuf[pl.ds(sg0 + j * _SB, _L)]
                for u in range(1, _SB // _L):
                    m = jnp.maximum(m, rowbuf[pl.ds(sg0 + j * _SB + u * _L, _L)])
                sum2[pl.ds(sgi * (_SG // _SB) * _L + j * _L, _L)] = m
                macc = m if macc is None else jnp.maximum(macc, m)
            sum1[pl.ds(sgi * _L, _L)] = macc

    w0[...] = jnp.full((_L,), _NEG, jnp.float32)
    w1[...] = jnp.full((_L,), _NEG, jnp.float32)
    tref[...] = jnp.full((_L,), _NEG, jnp.float32)

    def vinsert(v):
        sv = plsc.sort_key_val(v, v, descending=True)[0]
        rv = lax.rev(sv, (0,))
        x0 = w0[...]
        hi = jnp.maximum(x0, rv)
        lo = jnp.minimum(x0, rv)
        w0[...] = plsc.sort_key_val(hi, hi, descending=True)[0]
        lo_s = plsc.sort_key_val(lo, lo, descending=True)[0]
        rv2 = lax.rev(lo_s, (0,))
        x1 = w1[...]
        hi2 = jnp.maximum(x1, rv2)
        b = plsc.sort_key_val(hi2, hi2, descending=True)[0]
        w1[...] = b
        tref[...] = _bcast(jnp.min(b))

    @pl.loop(0, _NSG * _L, step=_L)
    def _(i):
        v = sum1[pl.ds(i, _L)]

        @pl.when(jnp.any(v > tref[...]))
        def _():
            vinsert(v)

    t0 = jnp.min(w1[...])
    t0m = t0 - (jnp.abs(t0) * jnp.float32(2e-6) + jnp.float32(1e-37))
    t0ref[...] = _bcast(t0m)
    tref[...] = _bcast(t0m)

    s0v[...] = jnp.full((_L,), _NEG, jnp.float32)
    s1v[...] = jnp.full((_L,), _NEG, jnp.float32)
    s0i[...] = jnp.zeros((_L,), jnp.int32)
    s1i[...] = jnp.zeros((_L,), jnp.int32)

    lane = lax.iota(jnp.int32, _L)

    def insert(v, base):
        giv = lane + base
        sv, si = plsc.sort_key_val(v, giv, descending=True)
        a_v, a_i, lo_v, lo_i = _merge16(s0v[...], s0i[...], sv, si)
        s0v[...] = a_v
        s0i[...] = a_i
        b_v, b_i, _, _ = _merge16(s1v[...], s1i[...], lo_v, lo_i)
        s1v[...] = b_v
        s1i[...] = b_i
        tref[...] = jnp.maximum(_bcast(jnp.min(b_v)), t0ref[...])

    @pl.loop(0, _NSG)
    def _(sg):
        sv = sum1[pl.ds(sg * _L, _L)]

        @pl.when(jnp.any(sv > tref[...]))
        def _():
            @pl.loop(0, _SG // _SB)
            def _(j):
                m8 = sum2[pl.ds(sg * (_SG // _SB) * _L + j * _L, _L)]

                @pl.when(jnp.any(m8 > tref[...]))
                def _():
                    @pl.loop(0, _SB, step=_L)
                    def _(u):
                        off = sg * _SG + j * _SB + u
                        v = rowbuf[pl.ds(off, _L)]

                        @pl.when(jnp.any(v > tref[...]))
                        def _():
                            insert(v, off)

    idxr[pl.ds(0, _L)] = s0i[...]
    idxr[pl.ds(_L, _L)] = s1i[...]
    for h in range(4):
        pltpu.async_copy(
            keys_hbm.at[idxr.at[pl.ds(h * 8, 8)]], gbuf, gsem).wait()
        pltpu.async_copy(
            gbuf, nbr_hbm.at[pl.ds(wid * TOPK + h * 8, 8)], osem).wait()


def _sc_compiler_params():
    cp = pltpu.CompilerParams()
    if "needs_layout_passes" in pltpu.CompilerParams.__dataclass_fields__:
        cp = dataclasses.replace(cp, needs_layout_passes=False)
    return cp


def _sc_topk_gather(sims_flat, keys):
    mesh = plsc.VectorSubcoreMesh(core_axis_name="c", subcore_axis_name="s")
    kern = functools.partial(
        pl.kernel,
        compiler_params=_sc_compiler_params(),
        out_type=jax.ShapeDtypeStruct((B * TOPK, D), jnp.float32),
        mesh=mesh,
        scratch_types=[
            pltpu.VMEM((_KPAD,), jnp.float32),
            pltpu.VMEM((_NSG * _L,), jnp.float32),
            pltpu.VMEM((_KPAD // _SB * _L,), jnp.float32),
            pltpu.VMEM((8, D), jnp.float32),
            pltpu.VMEM((TOPK,), jnp.int32),
            pltpu.VMEM((_L,), jnp.float32),
            pltpu.VMEM((_L,), jnp.int32),
            pltpu.VMEM((_L,), jnp.float32),
            pltpu.VMEM((_L,), jnp.int32),
            pltpu.VMEM((_L,), jnp.float32),
            pltpu.VMEM((_L,), jnp.float32),
            pltpu.VMEM((_L,), jnp.float32),
            pltpu.VMEM((_L,), jnp.float32),
        ] + [pltpu.SemaphoreType.DMA] * 9,
    )(_sc_body)
    return kern(sims_flat, keys)


def _fin_body(y0_ref, nbr_ref, m_ref, o_ref):
    w = jax.nn.softmax(m_ref[0, :]) * jnp.float32(1.0 / TOPK)
    nmean = jnp.sum(nbr_ref[...] * w[None, :, None], axis=1)
    pos = lax.broadcasted_iota(jnp.int32, (B, 8, D), 1)
    o_ref[...] = y0_ref[...] + jnp.where(pos == 0, nmean[:, None, :], 0.0)


def _finalize(y, nbr3, maskr):
    return pl.pallas_call(
        _fin_body,
        grid=(1,),
        in_specs=[
            pl.BlockSpec((B, 8, D), lambda i: (0, 0, 0)),
            pl.BlockSpec((B, TOPK, D), lambda i: (0, 0, 0)),
            pl.BlockSpec((1, TOPK), lambda i: (0, 0)),
        ],
        out_specs=pl.BlockSpec((B, 8, D), lambda i: (0, 0, 0)),
        out_shape=jax.ShapeDtypeStruct((B, S, D), jnp.float32),
        input_output_aliases={0: 0},
    )(y, nbr3, maskr)


def kernel(x, keys, W, b, mask):
    x2 = x.reshape(B * S, D)
    q = x[:, 0, :]
    sims = _sims(q, keys)
    y2 = _linear(x2, W, b.reshape(1, D))
    nbr = _sc_topk_gather(sims.reshape(B * _KPAD), keys)
    y = y2.reshape(B, S, D)
    return _finalize(y, nbr.reshape(B, TOPK, D), mask.reshape(1, TOPK))


# speedup vs baseline: 3.6944x; 1.0500x over previous
import dataclasses
import functools

import jax
import jax.numpy as jnp
from jax import lax
from jax.experimental import pallas as pl
from jax.experimental.pallas import tpu as pltpu
from jax.experimental.pallas import tpu_sc as plsc

B, S, D = 32, 128, 1024
K_KEYS = 100000
TOPK = 32


_KC = 2048
_NKC = (K_KEYS + _KC - 1) // _KC
_KPAD = _NKC * _KC


def _sims_body(q_ref, k_ref, o_ref):
    s = lax.dot_general(
        q_ref[...], k_ref[...], (((1,), (1,)), ((), ())),
        preferred_element_type=jnp.float32,
    )
    i = pl.program_id(0)

    @pl.when(i == _NKC - 1)
    def _():
        col = i * _KC + lax.broadcasted_iota(jnp.int32, (B, _KC), 1)
        o_ref[...] = jnp.where(col >= K_KEYS, jnp.float32(_NEG), s)

    @pl.when(i != _NKC - 1)
    def _():
        o_ref[...] = s


def _sims(q, keys):
    return pl.pallas_call(
        _sims_body,
        grid=(_NKC,),
        in_specs=[
            pl.BlockSpec((B, D), lambda i: (0, 0)),
            pl.BlockSpec((_KC, D), lambda i: (i, 0)),
        ],
        out_specs=pl.BlockSpec((B, _KC), lambda i: (0, i)),
        out_shape=jax.ShapeDtypeStruct((B, _KPAD), jnp.float32),
    )(q, keys)


_RB = 512


def _y_body(x_ref, w_ref, b_ref, o_ref):
    o_ref[...] = lax.dot_general(
        x_ref[...], w_ref[...], (((1,), (1,)), ((), ())),
        preferred_element_type=jnp.float32,
    ) + b_ref[...]


def _linear(x2, W, b2):
    return pl.pallas_call(
        _y_body,
        grid=(B * S // _RB,),
        in_specs=[
            pl.BlockSpec((_RB, D), lambda i: (i, 0)),
            pl.BlockSpec((D, D), lambda i: (0, 0)),
            pl.BlockSpec((1, D), lambda i: (0, 0)),
        ],
        out_specs=pl.BlockSpec((_RB, D), lambda i: (i, 0)),
        out_shape=jax.ShapeDtypeStruct((B * S, D), jnp.float32),
    )(x2, W, b2)


_L = 16
_SB = 256
_SG = 2048
_NSG = _KPAD // _SG
_CH = 7 * _SG
_NCHUNK = _KPAD // _CH

_NEG = -3.0e38


def _bcast(s):
    return jnp.zeros((_L,), jnp.float32) + s


def _merge16(xv, xi, yv, yi):
    rv = lax.rev(yv, (0,))
    ri = lax.rev(yi, (0,))
    m = xv >= rv
    hi = jnp.maximum(xv, rv)
    lo = jnp.minimum(xv, rv)
    hii = jnp.where(m, xi, ri)
    loi = jnp.where(m, ri, xi)
    hi_s, hii_s = plsc.sort_key_val(hi, hii, descending=True)
    lo_s, loi_s = plsc.sort_key_val(lo, loi, descending=True)
    return hi_s, hii_s, lo_s, loi_s


def _sc_body(sims_hbm, keys_hbm, nbr_hbm,
             rowbuf, sum1, sum2, gbuf0, gbuf1, idxr,
             s0v, s0i, s1v, s1i, tref, t0ref,
             csem0, csem1, csem2, csem3, csem4, csem5, csem6,
             gsem, osem0, osem1):
    wid = lax.axis_index("s") * 2 + lax.axis_index("c")
    csems = [csem0, csem1, csem2, csem3, csem4, csem5, csem6]

    cps = [
        pltpu.async_copy(
            sims_hbm.at[wid, pl.ds(c * _CH, _CH)],
            rowbuf.at[pl.ds(c * _CH, _CH)],
            csems[c],
        )
        for c in range(_NCHUNK)
    ]

    for c in range(_NCHUNK):
        cps[c].wait()

        @pl.loop(0, _CH // _SG)
        def _(s):
            sgi = c * (_CH // _SG) + s
            sg0 = sgi * _SG
            macc = None
            for j in range(_SG // _SB):
                m = rowbuf[pl.ds(sg0 + j * _SB, _L)]
                for u in range(1, _SB // _L):
                    m = jnp.maximum(m, rowbuf[pl.ds(sg0 + j * _SB + u * _L, _L)])
                sum2[pl.ds(sgi * (_SG // _SB) * _L + j * _L, _L)] = m
                macc = m if macc is None else jnp.maximum(macc, m)
            sum1[pl.ds(sgi * _L, _L)] = macc

    s0v[...] = jnp.full((_L,), _NEG, jnp.float32)
    s1v[...] = jnp.full((_L,), _NEG, jnp.float32)
    tref[...] = jnp.full((_L,), _NEG, jnp.float32)

    def vinsert(v):
        sv = plsc.sort_key_val(v, v, descending=True)[0]
        rv = lax.rev(sv, (0,))
        x0 = s0v[...]
        hi = jnp.maximum(x0, rv)
        lo = jnp.minimum(x0, rv)
        s0v[...] = plsc.sort_key_val(hi, hi, descending=True)[0]
        lo_s = plsc.sort_key_val(lo, lo, descending=True)[0]
        rv2 = lax.rev(lo_s, (0,))
        x1 = s1v[...]
        hi2 = jnp.maximum(x1, rv2)
        b = plsc.sort_key_val(hi2, hi2, descending=True)[0]
        s1v[...] = b
        tref[...] = _bcast(jnp.min(b))

    @pl.loop(0, _NSG * _L, step=_L)
    def _(i):
        v = sum1[pl.ds(i, _L)]

        @pl.when(jnp.any(v > tref[...]))
        def _():
            vinsert(v)

    t0 = jnp.min(s1v[...])
    t0m = t0 - (jnp.abs(t0) * jnp.float32(2e-6) + jnp.float32(1e-37))
    t0ref[...] = _bcast(t0m)
    tref[...] = _bcast(t0m)

    s0v[...] = jnp.full((_L,), _NEG, jnp.float32)
    s1v[...] = jnp.full((_L,), _NEG, jnp.float32)
    s0i[...] = jnp.zeros((_L,), jnp.int32)
    s1i[...] = jnp.zeros((_L,), jnp.int32)

    lane = lax.iota(jnp.int32, _L)

    def insert(v, base):
        giv = lane + base
        sv, si = plsc.sort_key_val(v, giv, descending=True)
        a_v, a_i, lo_v, lo_i = _merge16(s0v[...], s0i[...], sv, si)
        s0v[...] = a_v
        s0i[...] = a_i
        b_v, b_i, _, _ = _merge16(s1v[...], s1i[...], lo_v, lo_i)
        s1v[...] = b_v
        s1i[...] = b_i
        tref[...] = jnp.maximum(_bcast(jnp.min(b_v)), t0ref[...])

    @pl.loop(0, _NSG)
    def _(sg):
        sv = sum1[pl.ds(sg * _L, _L)]

        @pl.when(jnp.any(sv > tref[...]))
        def _():
            @pl.loop(0, _SG // _SB)
            def _(j):
                m8 = sum2[pl.ds(sg * (_SG // _SB) * _L + j * _L, _L)]

                @pl.when(jnp.any(m8 > tref[...]))
                def _():
                    @pl.loop(0, _SB, step=_L)
                    def _(u):
                        off = sg * _SG + j * _SB + u
                        v = rowbuf[pl.ds(off, _L)]

                        @pl.when(jnp.any(v > tref[...]))
                        def _():
                            insert(v, off)

    idxr[pl.ds(0, _L)] = s0i[...]
    idxr[pl.ds(_L, _L)] = s1i[...]
    gbufs = [gbuf0, gbuf1]
    osems = [osem0, osem1]
    outcps = [None, None]
    for h in range(4):
        b = h % 2
        if outcps[b] is not None:
            outcps[b].wait()
        pltpu.async_copy(
            keys_hbm.at[idxr.at[pl.ds(h * 8, 8)]], gbufs[b], gsem).wait()
        outcps[b] = pltpu.async_copy(
            gbufs[b], nbr_hbm.at[pl.ds(wid * TOPK + h * 8, 8)], osems[b])
    outcps[0].wait()
    outcps[1].wait()


def _sc_compiler_params():
    cp = pltpu.CompilerParams()
    if "needs_layout_passes" in pltpu.CompilerParams.__dataclass_fields__:
        cp = dataclasses.replace(cp, needs_layout_passes=False)
    return cp


def _sc_topk_gather(sims_flat, keys):
    mesh = plsc.VectorSubcoreMesh(core_axis_name="c", subcore_axis_name="s")
    kern = functools.partial(
        pl.kernel,
        compiler_params=_sc_compiler_params(),
        out_type=jax.ShapeDtypeStruct((B * TOPK, D), jnp.float32),
        mesh=mesh,
        scratch_types=[
            pltpu.VMEM((_KPAD,), jnp.float32),
            pltpu.VMEM((_NSG * _L,), jnp.float32),
            pltpu.VMEM((_KPAD // _SB * _L,), jnp.float32),
            pltpu.VMEM((8, D), jnp.float32),
            pltpu.VMEM((8, D), jnp.float32),
            pltpu.VMEM((TOPK,), jnp.int32),
            pltpu.VMEM((_L,), jnp.float32),
            pltpu.VMEM((_L,), jnp.int32),
            pltpu.VMEM((_L,), jnp.float32),
            pltpu.VMEM((_L,), jnp.int32),
            pltpu.VMEM((_L,), jnp.float32),
            pltpu.VMEM((_L,), jnp.float32),
        ] + [pltpu.SemaphoreType.DMA] * 10,
    )(_sc_body)
    return kern(sims_flat, keys)


def _fin_body(y0_ref, nbr_ref, m_ref, o_ref):
    w = jax.nn.softmax(m_ref[0, :]) * jnp.float32(1.0 / TOPK)
    nmean = jnp.sum(nbr_ref[...] * w[None, :, None], axis=1)
    pos = lax.broadcasted_iota(jnp.int32, (B, 8, D), 1)
    o_ref[...] = y0_ref[...] + jnp.where(pos == 0, nmean[:, None, :], 0.0)


def _finalize(y, nbr3, maskr):
    return pl.pallas_call(
        _fin_body,
        grid=(1,),
        in_specs=[
            pl.BlockSpec((B, 8, D), lambda i: (0, 0, 0)),
            pl.BlockSpec((B, TOPK, D), lambda i: (0, 0, 0)),
            pl.BlockSpec((1, TOPK), lambda i: (0, 0)),
        ],
        out_specs=pl.BlockSpec((B, 8, D), lambda i: (0, 0, 0)),
        out_shape=jax.ShapeDtypeStruct((B, S, D), jnp.float32),
        input_output_aliases={0: 0},
    )(y, nbr3, maskr)


def kernel(x, keys, W, b, mask):
    x2 = x.reshape(B * S, D)
    q = x[:, 0, :]
    sims = _sims(q, keys)
    y2 = _linear(x2, W, b.reshape(1, D))
    nbr = _sc_topk_gather(sims, keys)
    y = y2.reshape(B, S, D)
    return _finalize(y, nbr.reshape(B, TOPK, D), mask.reshape(1, TOPK))


# speedup vs baseline: 3.7197x; 1.0069x over previous
import dataclasses
import functools

import jax
import jax.numpy as jnp
from jax import lax
from jax.experimental import pallas as pl
from jax.experimental.pallas import tpu as pltpu
from jax.experimental.pallas import tpu_sc as plsc

B, S, D = 32, 128, 1024
K_KEYS = 100000
TOPK = 32

_KC = 1792
_KPAD = 100352
_HALF = _KPAD // 2
_NKCH = _HALF // _KC

_NEG = -3.0e38


def _sims_part(q, keys, h):
    def body(q_ref, k_ref, o_ref):
        s = lax.dot_general(
            q_ref[...], k_ref[...], (((1,), (1,)), ((), ())),
            preferred_element_type=jnp.float32,
        )
        if h == 1:
            i = pl.program_id(0)

            @pl.when(i == _NKCH - 1)
            def _():
                col = _HALF + i * _KC + lax.broadcasted_iota(
                    jnp.int32, (B, _KC), 1)
                o_ref[...] = jnp.where(col >= K_KEYS, jnp.float32(_NEG), s)

            @pl.when(i != _NKCH - 1)
            def _():
                o_ref[...] = s
        else:
            o_ref[...] = s

    return pl.pallas_call(
        body,
        grid=(_NKCH,),
        in_specs=[
            pl.BlockSpec((B, D), lambda i: (0, 0)),
            pl.BlockSpec((_KC, D), lambda i: (i + h * _NKCH, 0)),
        ],
        out_specs=pl.BlockSpec((B, _KC), lambda i: (0, i)),
        out_shape=jax.ShapeDtypeStruct((B, _HALF), jnp.float32),
    )(q, keys)


_RB = 512


def _y_body(x_ref, w_ref, b_ref, o_ref):
    o_ref[...] = lax.dot_general(
        x_ref[...], w_ref[...], (((1,), (1,)), ((), ())),
        preferred_element_type=jnp.float32,
    ) + b_ref[...]


def _linear(x2, W, b2):
    return pl.pallas_call(
        _y_body,
        grid=(B * S // _RB,),
        in_specs=[
            pl.BlockSpec((_RB, D), lambda i: (i, 0)),
            pl.BlockSpec((D, D), lambda i: (0, 0)),
            pl.BlockSpec((1, D), lambda i: (0, 0)),
        ],
        out_specs=pl.BlockSpec((_RB, D), lambda i: (i, 0)),
        out_shape=jax.ShapeDtypeStruct((B * S, D), jnp.float32),
    )(x2, W, b2)


_L = 16
_SB = 256
_SG = 1792
_NSGH = _HALF // _SG
_CH = 4 * _SG
_NCHUNK = _HALF // _CH


def _bcast(s):
    return jnp.zeros((_L,), jnp.float32) + s


def _merge16(xv, xi, yv, yi):
    rv = lax.rev(yv, (0,))
    ri = lax.rev(yi, (0,))
    m = xv >= rv
    hi = jnp.maximum(xv, rv)
    lo = jnp.minimum(xv, rv)
    hii = jnp.where(m, xi, ri)
    loi = jnp.where(m, ri, xi)
    hi_s, hii_s = plsc.sort_key_val(hi, hii, descending=True)
    lo_s, loi_s = plsc.sort_key_val(lo, loi, descending=True)
    return hi_s, hii_s, lo_s, loi_s


def _scan_half(h, sims_hbm, pv_hbm, pi_hbm,
               rowbuf, sum1, sum2, pvbuf, pibuf,
               s0v, s0i, s1v, s1i, tref, t0ref, csems, psem):
    wid = lax.axis_index("s") * 2 + lax.axis_index("c")
    hoff = h * _HALF

    pcp_v = pltpu.async_copy(pv_hbm.at[wid], pvbuf, psem)
    cps = [
        pltpu.async_copy(
            sims_hbm.at[wid, pl.ds(c * _CH, _CH)],
            rowbuf.at[pl.ds(c * _CH, _CH)],
            csems[c],
        )
        for c in range(_NCHUNK)
    ]
    pcp_i = pltpu.async_copy(pi_hbm.at[wid], pibuf, psem)

    for c in range(_NCHUNK):
        cps[c].wait()

        @pl.loop(0, _CH // _SG)
        def _(s):
            sgi = c * (_CH // _SG) + s
            sg0 = sgi * _SG
            macc = None
            for j in range(_SG // _SB):
                m = rowbuf[pl.ds(sg0 + j * _SB, _L)]
                for u in range(1, _SB // _L):
                    m = jnp.maximum(m, rowbuf[pl.ds(sg0 + j * _SB + u * _L, _L)])
                sum2[pl.ds(sgi * (_SG // _SB) * _L + j * _L, _L)] = m
                macc = m if macc is None else jnp.maximum(macc, m)
            sum1[pl.ds(sgi * _L, _L)] = macc

    s0v[...] = jnp.full((_L,), _NEG, jnp.float32)
    s1v[...] = jnp.full((_L,), _NEG, jnp.float32)
    tref[...] = jnp.full((_L,), _NEG, jnp.float32)

    def vinsert(v):
        sv = plsc.sort_key_val(v, v, descending=True)[0]
        rv = lax.rev(sv, (0,))
        x0 = s0v[...]
        hi = jnp.maximum(x0, rv)
        lo = jnp.minimum(x0, rv)
        s0v[...] = plsc.sort_key_val(hi, hi, descending=True)[0]
        lo_s = plsc.sort_key_val(lo, lo, descending=True)[0]
        rv2 = lax.rev(lo_s, (0,))
        x1 = s1v[...]
        hi2 = jnp.maximum(x1, rv2)
        b = plsc.sort_key_val(hi2, hi2, descending=True)[0]
        s1v[...] = b
        tref[...] = _bcast(jnp.min(b))

    @pl.loop(0, _NSGH * _L, step=_L)
    def _(i):
        v = sum1[pl.ds(i, _L)]

        @pl.when(jnp.any(v > tref[...]))
        def _():
            vinsert(v)

    pcp_v.wait()
    pcp_i.wait()
    prev_min = jnp.min(pvbuf[pl.ds(_L, _L)])
    t0 = jnp.min(s1v[...])
    t0m = t0 - (jnp.abs(t0) * jnp.float32(2e-6) + jnp.float32(1e-37))
    t0m = jnp.maximum(t0m, prev_min)
    t0ref[...] = _bcast(t0m)
    tref[...] = _bcast(t0m)

    s0v[...] = jnp.full((_L,), _NEG, jnp.float32)
    s1v[...] = jnp.full((_L,), _NEG, jnp.float32)
    s0i[...] = jnp.zeros((_L,), jnp.int32)
    s1i[...] = jnp.zeros((_L,), jnp.int32)

    lane = lax.iota(jnp.int32, _L)

    def insert(v, base):
        giv = lane + base
        sv, si = plsc.sort_key_val(v, giv, descending=True)
        a_v, a_i, lo_v, lo_i = _merge16(s0v[...], s0i[...], sv, si)
        s0v[...] = a_v
        s0i[...] = a_i
        b_v, b_i, _, _ = _merge16(s1v[...], s1i[...], lo_v, lo_i)
        s1v[...] = b_v
        s1i[...] = b_i
        tref[...] = jnp.maximum(_bcast(jnp.min(b_v)), t0ref[...])

    @pl.loop(0, _NSGH)
    def _(sg):
        sv = sum1[pl.ds(sg * _L, _L)]

        @pl.when(jnp.any(sv > tref[...]))
        def _():
            @pl.loop(0, _SG // _SB)
            def _(j):
                m16 = sum2[pl.ds(sg * (_SG // _SB) * _L + j * _L, _L)]

                @pl.when(jnp.any(m16 > tref[...]))
                def _():
                    @pl.loop(0, _SB, step=_L)
                    def _(u):
                        off = sg * _SG + j * _SB + u
                        v = rowbuf[pl.ds(off, _L)]

                        @pl.when(jnp.any(v > tref[...]))
                        def _():
                            insert(v, hoff + off)

    p0v = pvbuf[pl.ds(0, _L)]
    p1v = pvbuf[pl.ds(_L, _L)]
    p0i = pibuf[pl.ds(0, _L)]
    p1i = pibuf[pl.ds(_L, _L)]
    m0v, m0i, r0v, r0i = _merge16(p0v, p0i, s0v[...], s0i[...])
    u_v, u_i, _, _ = _merge16(r0v, r0i, s1v[...], s1i[...])
    m1v, m1i, _, _ = _merge16(u_v, u_i, p1v, p1i)
    s0v[...] = m0v
    s0i[...] = m0i
    s1v[...] = m1v
    s1i[...] = m1i
    return wid


def _sc_compiler_params():
    cp = pltpu.CompilerParams()
    if "needs_layout_passes" in pltpu.CompilerParams.__dataclass_fields__:
        cp = dataclasses.replace(cp, needs_layout_passes=False)
    return cp


_SC_SCRATCH = [
    pltpu.VMEM((_HALF,), jnp.float32),
    pltpu.VMEM((_NSGH * _L,), jnp.float32),
    pltpu.VMEM((_HALF // _SB * _L,), jnp.float32),
    pltpu.VMEM((TOPK,), jnp.float32),
    pltpu.VMEM((TOPK,), jnp.int32),
    pltpu.VMEM((_L,), jnp.float32),
    pltpu.VMEM((_L,), jnp.int32),
    pltpu.VMEM((_L,), jnp.float32),
    pltpu.VMEM((_L,), jnp.int32),
    pltpu.VMEM((_L,), jnp.float32),
    pltpu.VMEM((_L,), jnp.float32),
]


def _sc_part_body(sims_hbm, pv_hbm, pi_hbm, vout_hbm, iout_hbm,
                  rowbuf, sum1, sum2, pvbuf, pibuf,
                  s0v, s0i, s1v, s1i, tref, t0ref, valbuf, idxbuf,
                  csem0, csem1, csem2, csem3, csem4, csem5, csem6,
                  psem, osem):
    csems = [csem0, csem1, csem2, csem3, csem4, csem5, csem6]
    wid = _scan_half(0, sims_hbm, pv_hbm, pi_hbm,
                     rowbuf, sum1, sum2, pvbuf, pibuf,
                     s0v, s0i, s1v, s1i, tref, t0ref, csems, psem)
    valbuf[pl.ds(0, _L)] = s0v[...]
    valbuf[pl.ds(_L, _L)] = s1v[...]
    idxbuf[pl.ds(0, _L)] = s0i[...]
    idxbuf[pl.ds(_L, _L)] = s1i[...]
    pltpu.async_copy(valbuf, vout_hbm.at[wid], osem).wait()
    pltpu.async_copy(idxbuf, iout_hbm.at[wid], osem).wait()


def _sc_final_body(sims_hbm, pv_hbm, pi_hbm, keys_hbm, nbr_hbm,
                   rowbuf, sum1, sum2, pvbuf, pibuf,
                   s0v, s0i, s1v, s1i, tref, t0ref, idxr, gbuf0, gbuf1,
                   csem0, csem1, csem2, csem3, csem4, csem5, csem6,
                   psem, gsem, osem0, osem1):
    csems = [csem0, csem1, csem2, csem3, csem4, csem5, csem6]
    wid = _scan_half(1, sims_hbm, pv_hbm, pi_hbm,
                     rowbuf, sum1, sum2, pvbuf, pibuf,
                     s0v, s0i, s1v, s1i, tref, t0ref, csems, psem)

    idxr[pl.ds(0, _L)] = s0i[...]
    idxr[pl.ds(_L, _L)] = s1i[...]
    gbufs = [gbuf0, gbuf1]
    osems = [osem0, osem1]
    outcps = [None, None]
    for g in range(4):
        bb = g % 2
        if outcps[bb] is not None:
            outcps[bb].wait()
        pltpu.async_copy(
            keys_hbm.at[idxr.at[pl.ds(g * 8, 8)]], gbufs[bb], gsem).wait()
        outcps[bb] = pltpu.async_copy(
            gbufs[bb], nbr_hbm.at[pl.ds(wid * TOPK + g * 8, 8)], osems[bb])
    outcps[0].wait()
    outcps[1].wait()


def _sc_part(sims_h, pv, pi):
    mesh = plsc.VectorSubcoreMesh(core_axis_name="c", subcore_axis_name="s")
    kern = functools.partial(
        pl.kernel,
        compiler_params=_sc_compiler_params(),
        out_type=[
            jax.ShapeDtypeStruct((B, TOPK), jnp.float32),
            jax.ShapeDtypeStruct((B, TOPK), jnp.int32),
        ],
        mesh=mesh,
        scratch_types=_SC_SCRATCH + [
            pltpu.VMEM((TOPK,), jnp.float32),
            pltpu.VMEM((TOPK,), jnp.int32),
        ] + [pltpu.SemaphoreType.DMA] * 9,
    )(_sc_part_body)
    return kern(sims_h, pv, pi)


def _sc_final(sims_h, pv, pi, keys):
    mesh = plsc.VectorSubcoreMesh(core_axis_name="c", subcore_axis_name="s")
    kern = functools.partial(
        pl.kernel,
        compiler_params=_sc_compiler_params(),
        out_type=jax.ShapeDtypeStruct((B * TOPK, D), jnp.float32),
        mesh=mesh,
        scratch_types=_SC_SCRATCH + [
            pltpu.VMEM((TOPK,), jnp.int32),
            pltpu.VMEM((8, D), jnp.float32),
            pltpu.VMEM((8, D), jnp.float32),
        ] + [pltpu.SemaphoreType.DMA] * 11,
    )(_sc_final_body)
    return kern(sims_h, pv, pi, keys)


def _fin_body(y0_ref, nbr_ref, m_ref, o_ref):
    w = jax.nn.softmax(m_ref[0, :]) * jnp.float32(1.0 / TOPK)
    nmean = jnp.sum(nbr_ref[...] * w[None, :, None], axis=1)
    pos = lax.broadcasted_iota(jnp.int32, (B, 8, D), 1)
    o_ref[...] = y0_ref[...] + jnp.where(pos == 0, nmean[:, None, :], 0.0)


def _finalize(y, nbr3, maskr):
    return pl.pallas_call(
        _fin_body,
        grid=(1,),
        in_specs=[
            pl.BlockSpec((B, 8, D), lambda i: (0, 0, 0)),
            pl.BlockSpec((B, TOPK, D), lambda i: (0, 0, 0)),
            pl.BlockSpec((1, TOPK), lambda i: (0, 0)),
        ],
        out_specs=pl.BlockSpec((B, 8, D), lambda i: (0, 0, 0)),
        out_shape=jax.ShapeDtypeStruct((B, S, D), jnp.float32),
        input_output_aliases={0: 0},
    )(y, nbr3, maskr)


def kernel(x, keys, W, b, mask):
    x2 = x.reshape(B * S, D)
    q = x[:, 0, :]
    pv0 = jnp.full((B, TOPK), _NEG, jnp.float32)
    pi0 = jnp.zeros((B, TOPK), jnp.int32)
    sims_a = _sims_part(q, keys, 0)
    st_v, st_i = _sc_part(sims_a, pv0, pi0)
    sims_b = _sims_part(q, keys, 1)
    y2 = _linear(x2, W, b.reshape(1, D))
    nbr = _sc_final(sims_b, st_v, st_i, keys)
    y = y2.reshape(B, S, D)
    return _finalize(y, nbr.reshape(B, TOPK, D), mask.reshape(1, TOPK))
